# baseline (speedup 1.0000x reference)
import functools

import jax
import jax.numpy as jnp
from jax import lax
from jax.experimental import pallas as pl
from jax.experimental.pallas import tpu as pltpu
from jax.experimental.pallas import tpu_sc as plsc

_D = 128
_EOI = 99999
_NC = 2
_NS = 16
_NW = _NC * _NS
_C = 128
_CH = 50
_BPW = _C * _CH


def _embed_call(idx3, weight, eoi_embedding):
    B = _NW * _BPW
    mesh = plsc.VectorSubcoreMesh(core_axis_name="c", subcore_axis_name="s")

    @functools.partial(
        pl.kernel,
        mesh=mesh,
        out_type=jax.ShapeDtypeStruct((B, _D), jnp.float32),
        compiler_params=pltpu.CompilerParams(needs_layout_passes=False),
        scratch_types=[
            pltpu.VMEM((_CH, _C), jnp.int32),
            pltpu.VMEM((_D,), jnp.float32),
            pltpu.VMEM((_C, _D), jnp.float32),
            pltpu.VMEM((_C, _D), jnp.float32),
            pltpu.SemaphoreType.DMA,
            pltpu.SemaphoreType.DMA,
            pltpu.SemaphoreType.DMA,
            pltpu.SemaphoreType.DMA,
        ],
    )
    def emb(idx_hbm, table_hbm, eoi_hbm, out_hbm,
            idx_v, eoi_v, buf0, buf1, g0, g1, s0, s1):
        wid = lax.axis_index("s") * _NC + lax.axis_index("c")
        row_base = wid * _BPW

        pltpu.sync_copy(idx_hbm.at[wid], idx_v)
        pltpu.sync_copy(eoi_hbm, eoi_v)

        def start_gather(buf, gsem, c):
            pltpu.async_copy(table_hbm.at[idx_v.at[c]], buf, gsem)

        def wait_gather(buf, gsem, c):
            pltpu.make_async_copy(table_hbm.at[idx_v.at[c]], buf, gsem).wait()

        def fixup(buf, c):
            m = idx_v[c, pl.ds(0, 16)] == _EOI
            for g in range(1, _C // 16):
                m = jnp.logical_or(m, idx_v[c, pl.ds(g * 16, 16)] == _EOI)
            dirty = plsc.all_reduce_population_count(m)[0] > 0

            @pl.when(dirty)
            def _():
                def grp(g, carry):
                    ivec = idx_v[c, pl.ds(g * 16, 16)]

                    @pl.when(
                        plsc.all_reduce_population_count(ivec == _EOI)[0] > 0)
                    def _():
                        for l in range(16):
                            @pl.when(ivec[l] == _EOI)
                            def _():
                                for j in range(_D // 16):
                                    buf[g * 16 + l, pl.ds(j * 16, 16)] = (
                                        eoi_v[pl.ds(j * 16, 16)])
                    return carry
                lax.fori_loop(0, _C // 16, grp, 0)

        def start_scatter(buf, ssem, c):
            pltpu.async_copy(buf, out_hbm.at[pl.ds(row_base + c * _C, _C)], ssem)

        def wait_scatter(buf, ssem, c):
            pltpu.make_async_copy(
                buf, out_hbm.at[pl.ds(row_base + c * _C, _C)], ssem).wait()

        bufs = ((buf0, g0, s0), (buf1, g1, s1))

        start_gather(buf0, g0, 0)
        start_gather(buf1, g1, 1)

        def body(i, carry):
            k = i * 2
            for bi, (buf, gsem, ssem) in enumerate(bufs):
                c = k + bi
                wait_gather(buf, gsem, c)
                fixup(buf, c)
                start_scatter(buf, ssem, c)
                wait_scatter(buf, ssem, c)

                @pl.when(c + 2 < _CH)
                def _():
                    start_gather(buf, gsem, c + 2)
            return carry

        lax.fori_loop(0, _CH // 2, body, 0)

    return emb(idx3, weight, eoi_embedding)


def kernel(input_ids, weight, eoi_embedding):
    n_batch, n_tok = input_ids.shape
    ids = input_ids.T.reshape(-1).astype(jnp.int32)
    idx3 = ids.reshape(_NW, _CH, _C)
    out = _embed_call(idx3, weight.astype(jnp.float32),
                      eoi_embedding.astype(jnp.float32))
    return out.reshape(n_tok, n_batch, _D).transpose(1, 0, 2)


# speedup vs baseline: 1.0393x; 1.0393x over previous
import functools

import jax
import jax.numpy as jnp
from jax import lax
from jax.experimental import pallas as pl
from jax.experimental.pallas import tpu as pltpu
from jax.experimental.pallas import tpu_sc as plsc

_D = 128
_EOI = 99999
_NC = 2
_NS = 16
_NW = _NC * _NS
_C = 128
_CH = 50
_BPW = _C * _CH


def _embed_call(idx3, weight, eoi_embedding):
    B = _NW * _BPW
    mesh = plsc.VectorSubcoreMesh(core_axis_name="c", subcore_axis_name="s")

    @functools.partial(
        pl.kernel,
        mesh=mesh,
        out_type=jax.ShapeDtypeStruct((B, _D), jnp.float32),
        compiler_params=pltpu.CompilerParams(needs_layout_passes=False),
        scratch_types=[
            pltpu.VMEM((_CH, _C), jnp.int32),
            pltpu.VMEM((_D,), jnp.float32),
            pltpu.VMEM((4, _C, _D), jnp.float32),
            pltpu.SemaphoreType.DMA,
            pltpu.SemaphoreType.DMA,
        ],
    )
    def emb(idx_hbm, table_hbm, eoi_hbm, out_hbm,
            idx_v, eoi_v, ring, gsem, ssem):
        wid = lax.axis_index("s") * _NC + lax.axis_index("c")
        row_base = wid * _BPW

        pltpu.sync_copy(idx_hbm.at[wid], idx_v)
        pltpu.sync_copy(eoi_hbm, eoi_v)

        def start_gather(c):
            pltpu.async_copy(
                table_hbm.at[idx_v.at[c]], ring.at[lax.rem(c, 4)], gsem)

        def wait_gather(c):
            pltpu.make_async_copy(
                table_hbm.at[idx_v.at[c]], ring.at[lax.rem(c, 4)], gsem).wait()

        def fixup(c):
            b = lax.rem(c, 4)
            m = idx_v[c, pl.ds(0, 16)] == _EOI
            for g in range(1, _C // 16):
                m = jnp.logical_or(m, idx_v[c, pl.ds(g * 16, 16)] == _EOI)
            dirty = plsc.all_reduce_population_count(m)[0] > 0

            @pl.when(dirty)
            def _():
                def grp(g, carry):
                    ivec = idx_v[c, pl.ds(g * 16, 16)]

                    @pl.when(
                        plsc.all_reduce_population_count(ivec == _EOI)[0] > 0)
                    def _():
                        for l in range(16):
                            @pl.when(ivec[l] == _EOI)
                            def _():
                                for j in range(_D // 16):
                                    ring[b, g * 16 + l, pl.ds(j * 16, 16)] = (
                                        eoi_v[pl.ds(j * 16, 16)])
                    return carry
                lax.fori_loop(0, _C // 16, grp, 0)

        def start_scatter(c):
            pltpu.async_copy(ring.at[lax.rem(c, 4)],
                             out_hbm.at[pl.ds(row_base + c * _C, _C)], ssem)

        def wait_scatter(c):
            pltpu.make_async_copy(
                ring.at[lax.rem(c, 4)],
                out_hbm.at[pl.ds(row_base + c * _C, _C)], ssem).wait()

        start_gather(0)
        start_gather(1)

        def body(c, carry):
            wait_gather(c)
            fixup(c)
            start_scatter(c)

            @pl.when(c >= 2)
            def _():
                wait_scatter(c - 2)

            @pl.when(c + 2 < _CH)
            def _():
                start_gather(c + 2)
            return carry

        lax.fori_loop(0, _CH, body, 0)

        wait_scatter(_CH - 2)
        wait_scatter(_CH - 1)

    return emb(idx3, weight, eoi_embedding)


def kernel(input_ids, weight, eoi_embedding):
    n_batch, n_tok = input_ids.shape
    ids = input_ids.T.reshape(-1).astype(jnp.int32)
    idx3 = ids.reshape(_NW, _CH, _C)
    out = _embed_call(idx3, weight.astype(jnp.float32),
                      eoi_embedding.astype(jnp.float32))
    return out.reshape(n_tok, n_batch, _D).transpose(1, 0, 2)
